# baseline (speedup 1.0000x reference)
import functools

import jax
import jax.numpy as jnp
from jax import lax
from jax.experimental import pallas as pl
from jax.experimental.pallas import tpu as pltpu
from jax.experimental.pallas import tpu_sc as plsc

_N = 10000
_E = 320000
_F_IN = 128
_HEADS = 8
_HID = 16
_C = 40
_C_PAD = 48

_NC = 2
_NS = 16
_NW = _NC * _NS
_EPT = _E // _NW
_CHUNK = 50
_NCH = _EPT // _CHUNK
_RS = 624
_TAIL = _N - _NS * _RS

_BR = 1000


def _pre1_body(x_ref, w_ref, asm_ref, adm_ref, hh_ref, adp_ref):
    h = jnp.dot(x_ref[...], w_ref[...], preferred_element_type=jnp.float32)
    asp = jnp.dot(h, asm_ref[...], preferred_element_type=jnp.float32)
    hh_ref[...] = jnp.concatenate([h, asp], axis=1)
    adp_ref[...] = jnp.dot(h, adm_ref[...], preferred_element_type=jnp.float32)


def _pre1(x, W, Asm, Adm):
    return pl.pallas_call(
        _pre1_body,
        grid=(_N // _BR,),
        in_specs=[
            pl.BlockSpec((_BR, _F_IN), lambda i: (i, 0)),
            pl.BlockSpec((_F_IN, 128), lambda i: (0, 0)),
            pl.BlockSpec((128, 16), lambda i: (0, 0)),
            pl.BlockSpec((128, 16), lambda i: (0, 0)),
        ],
        out_specs=[
            pl.BlockSpec((_BR, 144), lambda i: (i, 0)),
            pl.BlockSpec((_BR, 16), lambda i: (i, 0)),
        ],
        out_shape=[
            jax.ShapeDtypeStruct((_N, 144), jnp.float32),
            jax.ShapeDtypeStruct((_N, 16), jnp.float32),
        ],
    )(x, W, Asm, Adm)


def _comb_body(accw_ref, hh_ref, adp_ref, b_ref, g_ref,
               be_ref, e8_ref, wn_ref, asn_ref, adn_ref,
               hhn_ref, adpn_ref):
    hh = hh_ref[...]
    h = hh[:, 0:128]
    asp = hh[:, 128:144]
    e = asp + adp_ref[...]
    wself = jnp.exp(jnp.where(e >= 0.0, e, 0.2 * e))
    accw = accw_ref[0] + accw_ref[1]
    den16 = accw[:, 128:144] + wself
    e8 = e8_ref[...]
    den = jnp.dot(den16, e8, preferred_element_type=jnp.float32)
    acc = (accw[:, 0:128]
           + jnp.dot(wself, e8, preferred_element_type=jnp.float32) * h)
    out = acc / den + b_ref[0:1, :]
    out = jnp.where(out > 0.0, out, jnp.exp(out) - 1.0)
    out = out * g_ref[0:1, :] + be_ref[0:1, :]
    hn = jnp.dot(out, wn_ref[...], preferred_element_type=jnp.float32)
    aspn = jnp.dot(hn, asn_ref[...], preferred_element_type=jnp.float32)
    hhn_ref[...] = jnp.concatenate([hn, aspn], axis=1)
    adpn_ref[...] = jnp.dot(hn, adn_ref[...], preferred_element_type=jnp.float32)


def _combine_pre(accw, hh, adp, b8, g8, be8, E8, Wn, Asn, Adn, Fn):
    return pl.pallas_call(
        _comb_body,
        grid=(_N // _BR,),
        in_specs=[
            pl.BlockSpec((2, _BR, 144), lambda i: (0, i, 0)),
            pl.BlockSpec((_BR, 144), lambda i: (i, 0)),
            pl.BlockSpec((_BR, 16), lambda i: (i, 0)),
            pl.BlockSpec((8, 128), lambda i: (0, 0)),
            pl.BlockSpec((8, 128), lambda i: (0, 0)),
            pl.BlockSpec((8, 128), lambda i: (0, 0)),
            pl.BlockSpec((16, 128), lambda i: (0, 0)),
            pl.BlockSpec((128, Fn), lambda i: (0, 0)),
            pl.BlockSpec((Fn, 16), lambda i: (0, 0)),
            pl.BlockSpec((Fn, 16), lambda i: (0, 0)),
        ],
        out_specs=[
            pl.BlockSpec((_BR, Fn + 16), lambda i: (i, 0)),
            pl.BlockSpec((_BR, 16), lambda i: (i, 0)),
        ],
        out_shape=[
            jax.ShapeDtypeStruct((_N, Fn + 16), jnp.float32),
            jax.ShapeDtypeStruct((_N, 16), jnp.float32),
        ],
    )(accw, hh, adp, b8, g8, be8, E8, Wn, Asn, Adn)


def _final_body(accw_ref, hh_ref, adp_ref, b_ref, e3_ref, out_ref):
    hh = hh_ref[...]
    h = hh[:, 0:_C_PAD]
    asp = hh[:, _C_PAD:_C_PAD + 16]
    e = asp + adp_ref[...]
    wself = jnp.exp(jnp.where(e >= 0.0, e, 0.2 * e))
    accw = accw_ref[0] + accw_ref[1]
    den16 = accw[:, _C_PAD:_C_PAD + 16] + wself
    e3 = e3_ref[...]
    den = jnp.dot(den16, e3, preferred_element_type=jnp.float32)
    acc = (accw[:, 0:_C_PAD]
           + jnp.dot(wself, e3, preferred_element_type=jnp.float32) * h)
    z = acc / den + b_ref[0:1, :]
    col = lax.broadcasted_iota(jnp.int32, (_BR, _C_PAD), 1)
    zm = jnp.where(col < _C, z, -1e30)
    m = jnp.max(zm, axis=1, keepdims=True)
    s = jnp.sum(jnp.exp(zm - m), axis=1, keepdims=True)
    out_ref[...] = z - (m + jnp.log(s))


def _final(accw, hh, adp, b8, E3):
    return pl.pallas_call(
        _final_body,
        grid=(_N // _BR,),
        in_specs=[
            pl.BlockSpec((2, _BR, _C_PAD + 16), lambda i: (0, i, 0)),
            pl.BlockSpec((_BR, _C_PAD + 16), lambda i: (i, 0)),
            pl.BlockSpec((_BR, 16), lambda i: (i, 0)),
            pl.BlockSpec((8, _C_PAD), lambda i: (0, 0)),
            pl.BlockSpec((16, _C_PAD), lambda i: (0, 0)),
        ],
        out_specs=[pl.BlockSpec((_BR, _C_PAD), lambda i: (i, 0))],
        out_shape=[jax.ShapeDtypeStruct((_N, _C_PAD), jnp.float32)],
    )(accw, hh, adp, b8, E3)[0]


def _build_sc(F, heads):
    FW = F + 16
    mesh = plsc.VectorSubcoreMesh(core_axis_name="c", subcore_axis_name="s")

    @functools.partial(
        pl.kernel,
        out_type=[jax.ShapeDtypeStruct((_NC, _N, FW), jnp.float32)],
        mesh=mesh,
        compiler_params=pltpu.CompilerParams(use_tc_tiling_on_sc=False),
        scratch_types=[
            pltpu.VMEM((_NCH, _CHUNK), jnp.int32),
            pltpu.VMEM((_NCH, _CHUNK), jnp.int32),
            pltpu.VMEM((_CHUNK, FW), jnp.float32),
            pltpu.VMEM((_CHUNK, FW), jnp.float32),
            pltpu.VMEM((_CHUNK, 16), jnp.float32),
            pltpu.VMEM((_CHUNK, 16), jnp.float32),
            pltpu.SemaphoreType.DMA,
            pltpu.SemaphoreType.DMA,
            pltpu.SemaphoreType.DMA,
            pltpu.SemaphoreType.DMA,
            pltpu.VMEM_SHARED((_N, FW), jnp.float32),
        ],
    )
    def sck(src_hbm, dst_hbm, hh_hbm, adp_hbm, zrow_hbm,
            acc_out,
            sidx_all, didx_all, rows0, rows1, adr0, adr1,
            gsem0, gsem1, ssem0, ssem1, acc_sh):
        c = lax.axis_index("c")
        s = lax.axis_index("s")
        wid = c * _NS + s
        rows = (rows0, rows1)
        adr = (adr0, adr1)
        gsem = (gsem0, gsem1)
        ssem = (ssem0, ssem1)

        pltpu.sync_copy(zrow_hbm, acc_sh.at[pl.ds(s * _RS, _RS)])

        @pl.when(s == _NS - 1)
        def _init_tail():
            pltpu.sync_copy(zrow_hbm.at[pl.ds(0, _TAIL)],
                            acc_sh.at[pl.ds(_NS * _RS, _TAIL)])

        pltpu.sync_copy(src_hbm.at[wid], sidx_all)
        pltpu.sync_copy(dst_hbm.at[wid], didx_all)
        plsc.subcore_barrier()

        def issue_gather(b, ci):
            pltpu.async_copy(hh_hbm.at[sidx_all.at[ci]], rows[b], gsem[b])
            pltpu.async_copy(adp_hbm.at[didx_all.at[ci]], adr[b], gsem[b])

        def wait_gather(b):
            pltpu.make_async_copy(hh_hbm.at[sidx_all.at[0]], rows[b], gsem[b]).wait()
            pltpu.make_async_copy(adp_hbm.at[didx_all.at[0]], adr[b], gsem[b]).wait()

        def issue_scatter(b, ci):
            pltpu.async_copy(rows[b], acc_sh.at[didx_all.at[ci]], ssem[b], add=True)

        def wait_scatter(b):
            pltpu.make_async_copy(rows[b], acc_sh.at[didx_all.at[0]], ssem[b]).wait()

        def compute(b):
            @plsc.parallel_loop(0, _CHUNK, 1, unroll=4)
            def edge_body(i):
                e = rows[b][i, pl.ds(F, 16)] + adr[b][i, :]
                w = jnp.exp(jnp.where(e >= 0.0, e, e * 0.2))
                rows[b][i, pl.ds(F, 16)] = w
                for k in range(F // 16):
                    hk = k if heads == 8 else 0
                    sw = w[hk]
                    rows[b][i, pl.ds(k * 16, 16)] = (
                        rows[b][i, pl.ds(k * 16, 16)] * sw)

        issue_gather(0, 0)

        def outer_body(g, carry):
            for b in (0, 1):
                ci = g * 2 + b
                nb = 1 - b

                @pl.when(ci + 1 < _NCH)
                def _prefetch():
                    @pl.when(ci >= 1)
                    def _recycle():
                        wait_scatter(nb)

                    issue_gather(nb, ci + 1)

                wait_gather(b)
                compute(b)
                issue_scatter(b, ci)
            return carry

        lax.fori_loop(0, _NCH // 2, outer_body, 0)
        wait_scatter(0)
        wait_scatter(1)
        plsc.subcore_barrier()

        pltpu.sync_copy(acc_sh.at[pl.ds(s * _RS, _RS)],
                        acc_out.at[c, pl.ds(s * _RS, _RS)])

        @pl.when(s == _NS - 1)
        def _out_tail():
            pltpu.sync_copy(acc_sh.at[pl.ds(_NS * _RS, _TAIL)],
                            acc_out.at[c, pl.ds(_NS * _RS, _TAIL)])

    return sck


_sc128 = _build_sc(128, 8)
_sc48 = _build_sc(_C_PAD, 1)


def _dup_head_mat(a):
    eye8 = jnp.eye(8, dtype=jnp.float32)
    m8 = (a[:, :, None] * eye8[:, None, :]).reshape(128, 8)
    return jnp.concatenate([m8, m8], axis=1)


def _single_head_mat(a):
    ap = jnp.concatenate([a.reshape(_C, 1),
                          jnp.zeros((_C_PAD - _C, 1), jnp.float32)], axis=0)
    return jnp.tile(ap, (1, 16))


def _row8(v, F):
    return jnp.tile(v.reshape(1, F), (8, 1))


def _impl(x, edge_index, W1, a_src1, a_dst1, b1, g1, be1,
          W2, a_src2, a_dst2, b2, g2, be2, W3, a_src3, a_dst3, b3):
    src = edge_index[0].reshape(_NW, _NCH, _CHUNK)
    dst = edge_index[1].reshape(_NW, _NCH, _CHUNK)
    bn_s = 1.0 / jnp.sqrt(jnp.float32(1.0 + 1e-5))

    As1 = _dup_head_mat(a_src1)
    Ad1 = _dup_head_mat(a_dst1)
    As2 = _dup_head_mat(a_src2)
    Ad2 = _dup_head_mat(a_dst2)
    As3 = _single_head_mat(a_src3)
    Ad3 = _single_head_mat(a_dst3)
    W3p = jnp.concatenate([W3, jnp.zeros((128, _C_PAD - _C), jnp.float32)], 1)
    b3p = jnp.concatenate([b3, jnp.zeros((_C_PAD - _C,), jnp.float32)])

    idx16 = jnp.arange(16, dtype=jnp.int32)
    E8 = (idx16[:, None] == (jnp.arange(128, dtype=jnp.int32) // 16)[None, :])
    E8 = E8.astype(jnp.float32)
    E3 = (idx16[:, None] == 0).astype(jnp.float32) * jnp.ones((1, _C_PAD), jnp.float32)

    zrow = jnp.zeros((_RS, 144), jnp.float32)
    zrow3 = jnp.zeros((_RS, _C_PAD + 16), jnp.float32)

    hh1, adp1 = _pre1(x, W1, As1, Ad1)
    accw1 = _sc128(src, dst, hh1, adp1, zrow)[0]
    hh2, adp2 = _combine_pre(
        accw1, hh1, adp1,
        _row8(b1, 128), _row8(g1 * bn_s, 128), _row8(be1, 128),
        E8, W2, As2, Ad2, 128)
    accw2 = _sc128(src, dst, hh2, adp2, zrow)[0]
    hh3, adp3 = _combine_pre(
        accw2, hh2, adp2,
        _row8(b2, 128), _row8(g2 * bn_s, 128), _row8(be2, 128),
        E8, W3p, As3, Ad3, _C_PAD)
    accw3 = _sc48(src, dst, hh3, adp3, zrow3)[0]
    outp = _final(accw3, hh3, adp3, _row8(b3p, _C_PAD), E3)
    return outp[:, :_C]


_impl_jit = jax.jit(_impl)


def kernel(x, edge_index, W1, a_src1, a_dst1, b1, g1, be1,
           W2, a_src2, a_dst2, b2, g2, be2, W3, a_src3, a_dst3, b3):
    return _impl_jit(x, edge_index, W1, a_src1, a_dst1, b1, g1, be1,
                     W2, a_src2, a_dst2, b2, g2, be2, W3, a_src3, a_dst3, b3)


# speedup vs baseline: 1.0179x; 1.0179x over previous
import functools

import jax
import jax.numpy as jnp
from jax import lax
from jax.experimental import pallas as pl
from jax.experimental.pallas import tpu as pltpu
from jax.experimental.pallas import tpu_sc as plsc

_N = 10000
_E = 320000
_F_IN = 128
_HEADS = 8
_HID = 16
_C = 40
_C_PAD = 48

_NC = 2
_NS = 16
_NW = _NC * _NS
_EPT = _E // _NW
_CHUNK = 50
_NCH = _EPT // _CHUNK
_RS = 624
_TAIL = _N - _NS * _RS

_BR = 1000


def _pre1_body(x_ref, w_ref, asm_ref, adm_ref, h_ref, asp_ref, adp_ref):
    h = jnp.dot(x_ref[...], w_ref[...], preferred_element_type=jnp.float32)
    h_ref[...] = h
    asp_ref[...] = jnp.dot(h, asm_ref[...], preferred_element_type=jnp.float32)
    adp_ref[...] = jnp.dot(h, adm_ref[...], preferred_element_type=jnp.float32)


def _pre1(x, W, Asm, Adm):
    return pl.pallas_call(
        _pre1_body,
        grid=(_N // _BR,),
        in_specs=[
            pl.BlockSpec((_BR, _F_IN), lambda i: (i, 0)),
            pl.BlockSpec((_F_IN, 128), lambda i: (0, 0)),
            pl.BlockSpec((128, 16), lambda i: (0, 0)),
            pl.BlockSpec((128, 16), lambda i: (0, 0)),
        ],
        out_specs=[
            pl.BlockSpec((_BR, 128), lambda i: (i, 0)),
            pl.BlockSpec((_BR, 16), lambda i: (i, 0)),
            pl.BlockSpec((_BR, 16), lambda i: (i, 0)),
        ],
        out_shape=[
            jax.ShapeDtypeStruct((_N, 128), jnp.float32),
            jax.ShapeDtypeStruct((_N, 16), jnp.float32),
            jax.ShapeDtypeStruct((_N, 16), jnp.float32),
        ],
    )(x, W, Asm, Adm)


def _comb_body(acc_ref, den_ref, h_ref, asp_ref, adp_ref, b_ref, g_ref,
               be_ref, e8_ref, wn_ref, asn_ref, adn_ref,
               hn_ref, aspn_ref, adpn_ref):
    e = asp_ref[...] + adp_ref[...]
    wself = jnp.exp(jnp.where(e >= 0.0, e, 0.2 * e))
    den16 = den_ref[0] + den_ref[1] + wself
    e8 = e8_ref[...]
    den = jnp.dot(den16, e8, preferred_element_type=jnp.float32)
    acc = (acc_ref[0] + acc_ref[1]
           + jnp.dot(wself, e8, preferred_element_type=jnp.float32) * h_ref[...])
    out = acc / den + b_ref[0:1, :]
    out = jnp.where(out > 0.0, out, jnp.exp(out) - 1.0)
    out = out * g_ref[0:1, :] + be_ref[0:1, :]
    hn = jnp.dot(out, wn_ref[...], preferred_element_type=jnp.float32)
    hn_ref[...] = hn
    aspn_ref[...] = jnp.dot(hn, asn_ref[...], preferred_element_type=jnp.float32)
    adpn_ref[...] = jnp.dot(hn, adn_ref[...], preferred_element_type=jnp.float32)


def _combine_pre(acc, den, h, asp, adp, b8, g8, be8, E8, Wn, Asn, Adn, Fn):
    F = h.shape[1]
    return pl.pallas_call(
        _comb_body,
        grid=(_N // _BR,),
        in_specs=[
            pl.BlockSpec((2, _BR, F), lambda i: (0, i, 0)),
            pl.BlockSpec((2, _BR, 16), lambda i: (0, i, 0)),
            pl.BlockSpec((_BR, F), lambda i: (i, 0)),
            pl.BlockSpec((_BR, 16), lambda i: (i, 0)),
            pl.BlockSpec((_BR, 16), lambda i: (i, 0)),
            pl.BlockSpec((8, F), lambda i: (0, 0)),
            pl.BlockSpec((8, F), lambda i: (0, 0)),
            pl.BlockSpec((8, F), lambda i: (0, 0)),
            pl.BlockSpec((16, F), lambda i: (0, 0)),
            pl.BlockSpec((F, Fn), lambda i: (0, 0)),
            pl.BlockSpec((Fn, 16), lambda i: (0, 0)),
            pl.BlockSpec((Fn, 16), lambda i: (0, 0)),
        ],
        out_specs=[
            pl.BlockSpec((_BR, Fn), lambda i: (i, 0)),
            pl.BlockSpec((_BR, 16), lambda i: (i, 0)),
            pl.BlockSpec((_BR, 16), lambda i: (i, 0)),
        ],
        out_shape=[
            jax.ShapeDtypeStruct((_N, Fn), jnp.float32),
            jax.ShapeDtypeStruct((_N, 16), jnp.float32),
            jax.ShapeDtypeStruct((_N, 16), jnp.float32),
        ],
    )(acc, den, h, asp, adp, b8, g8, be8, E8, Wn, Asn, Adn)


def _final_body(acc_ref, den_ref, h_ref, asp_ref, adp_ref, b_ref, e3_ref,
                out_ref):
    e = asp_ref[...] + adp_ref[...]
    wself = jnp.exp(jnp.where(e >= 0.0, e, 0.2 * e))
    den16 = den_ref[0] + den_ref[1] + wself
    e3 = e3_ref[...]
    den = jnp.dot(den16, e3, preferred_element_type=jnp.float32)
    acc = (acc_ref[0] + acc_ref[1]
           + jnp.dot(wself, e3, preferred_element_type=jnp.float32) * h_ref[...])
    z = acc / den + b_ref[0:1, :]
    col = lax.broadcasted_iota(jnp.int32, (_BR, _C_PAD), 1)
    zm = jnp.where(col < _C, z, -1e30)
    m = jnp.max(zm, axis=1, keepdims=True)
    s = jnp.sum(jnp.exp(zm - m), axis=1, keepdims=True)
    out_ref[...] = z - (m + jnp.log(s))


def _final(acc, den, h, asp, adp, b8, E3):
    return pl.pallas_call(
        _final_body,
        grid=(_N // _BR,),
        in_specs=[
            pl.BlockSpec((2, _BR, _C_PAD), lambda i: (0, i, 0)),
            pl.BlockSpec((2, _BR, 16), lambda i: (0, i, 0)),
            pl.BlockSpec((_BR, _C_PAD), lambda i: (i, 0)),
            pl.BlockSpec((_BR, 16), lambda i: (i, 0)),
            pl.BlockSpec((_BR, 16), lambda i: (i, 0)),
            pl.BlockSpec((8, _C_PAD), lambda i: (0, 0)),
            pl.BlockSpec((16, _C_PAD), lambda i: (0, 0)),
        ],
        out_specs=[pl.BlockSpec((_BR, _C_PAD), lambda i: (i, 0))],
        out_shape=[jax.ShapeDtypeStruct((_N, _C_PAD), jnp.float32)],
    )(acc, den, h, asp, adp, b8, E3)[0]


def _build_sc(F, heads):
    mesh = plsc.VectorSubcoreMesh(core_axis_name="c", subcore_axis_name="s")

    @functools.partial(
        pl.kernel,
        out_type=[
            jax.ShapeDtypeStruct((_NC, _N, F), jnp.float32),
            jax.ShapeDtypeStruct((_NC, _N, 16), jnp.float32),
        ],
        mesh=mesh,
        compiler_params=pltpu.CompilerParams(use_tc_tiling_on_sc=False),
        scratch_types=[
            pltpu.VMEM((_NCH, _CHUNK), jnp.int32),
            pltpu.VMEM((_NCH, _CHUNK), jnp.int32),
            pltpu.VMEM((_CHUNK, F), jnp.float32),
            pltpu.VMEM((_CHUNK, F), jnp.float32),
            pltpu.VMEM((_CHUNK, 16), jnp.float32),
            pltpu.VMEM((_CHUNK, 16), jnp.float32),
            pltpu.VMEM((_CHUNK, 16), jnp.float32),
            pltpu.VMEM((_CHUNK, 16), jnp.float32),
            pltpu.VMEM((_CHUNK, 16), jnp.float32),
            pltpu.VMEM((_CHUNK, 16), jnp.float32),
            pltpu.SemaphoreType.DMA,
            pltpu.SemaphoreType.DMA,
            pltpu.SemaphoreType.DMA,
            pltpu.SemaphoreType.DMA,
            pltpu.VMEM_SHARED((_N, F), jnp.float32),
            pltpu.VMEM_SHARED((_N, 16), jnp.float32),
        ],
    )
    def sck(src_hbm, dst_hbm, h_hbm, asp_hbm, adp_hbm, zrow_hbm, zden_hbm,
            acc_out, den_out,
            sidx_all, didx_all, rows0, rows1, asr0, asr1, adr0, adr1,
            wbuf0, wbuf1, gsem0, gsem1, ssem0, ssem1, acc_sh, den_sh):
        c = lax.axis_index("c")
        s = lax.axis_index("s")
        wid = c * _NS + s
        rows = (rows0, rows1)
        asr = (asr0, asr1)
        adr = (adr0, adr1)
        wbuf = (wbuf0, wbuf1)
        gsem = (gsem0, gsem1)
        ssem = (ssem0, ssem1)

        pltpu.sync_copy(zrow_hbm, acc_sh.at[pl.ds(s * _RS, _RS)])
        pltpu.sync_copy(zden_hbm, den_sh.at[pl.ds(s * _RS, _RS)])

        @pl.when(s == _NS - 1)
        def _init_tail():
            pltpu.sync_copy(zrow_hbm.at[pl.ds(0, _TAIL)],
                            acc_sh.at[pl.ds(_NS * _RS, _TAIL)])
            pltpu.sync_copy(zden_hbm.at[pl.ds(0, _TAIL)],
                            den_sh.at[pl.ds(_NS * _RS, _TAIL)])

        pltpu.sync_copy(src_hbm.at[wid], sidx_all)
        pltpu.sync_copy(dst_hbm.at[wid], didx_all)
        plsc.subcore_barrier()

        def issue_gather(b, ci):
            pltpu.async_copy(h_hbm.at[sidx_all.at[ci]], rows[b], gsem[b])
            pltpu.async_copy(asp_hbm.at[sidx_all.at[ci]], asr[b], gsem[b])
            pltpu.async_copy(adp_hbm.at[didx_all.at[ci]], adr[b], gsem[b])

        def wait_gather(b):
            pltpu.make_async_copy(h_hbm.at[sidx_all.at[0]], rows[b], gsem[b]).wait()
            pltpu.make_async_copy(asp_hbm.at[sidx_all.at[0]], asr[b], gsem[b]).wait()
            pltpu.make_async_copy(adp_hbm.at[didx_all.at[0]], adr[b], gsem[b]).wait()

        def issue_scatter(b, ci):
            pltpu.async_copy(rows[b], acc_sh.at[didx_all.at[ci]], ssem[b], add=True)
            pltpu.async_copy(wbuf[b], den_sh.at[didx_all.at[ci]], ssem[b], add=True)

        def wait_scatter(b):
            pltpu.make_async_copy(rows[b], acc_sh.at[didx_all.at[0]], ssem[b]).wait()
            pltpu.make_async_copy(wbuf[b], den_sh.at[didx_all.at[0]], ssem[b]).wait()

        def compute(b):
            @plsc.parallel_loop(0, _CHUNK, 1, unroll=4)
            def edge_body(i):
                e = asr[b][i, :] + adr[b][i, :]
                w = jnp.exp(jnp.where(e >= 0.0, e, e * 0.2))
                wbuf[b][i, :] = w
                for k in range(F // 16):
                    hk = k if heads == 8 else 0
                    sw = w[hk]
                    rows[b][i, pl.ds(k * 16, 16)] = (
                        rows[b][i, pl.ds(k * 16, 16)] * sw)

        issue_gather(0, 0)

        def outer_body(g, carry):
            for b in (0, 1):
                ci = g * 2 + b
                nb = 1 - b

                @pl.when(ci + 1 < _NCH)
                def _prefetch():
                    @pl.when(ci >= 1)
                    def _recycle():
                        wait_scatter(nb)

                    issue_gather(nb, ci + 1)

                wait_gather(b)
                compute(b)
                issue_scatter(b, ci)
            return carry

        lax.fori_loop(0, _NCH // 2, outer_body, 0)
        wait_scatter(0)
        wait_scatter(1)
        plsc.subcore_barrier()

        pltpu.sync_copy(acc_sh.at[pl.ds(s * _RS, _RS)],
                        acc_out.at[c, pl.ds(s * _RS, _RS)])
        pltpu.sync_copy(den_sh.at[pl.ds(s * _RS, _RS)],
                        den_out.at[c, pl.ds(s * _RS, _RS)])

        @pl.when(s == _NS - 1)
        def _out_tail():
            pltpu.sync_copy(acc_sh.at[pl.ds(_NS * _RS, _TAIL)],
                            acc_out.at[c, pl.ds(_NS * _RS, _TAIL)])
            pltpu.sync_copy(den_sh.at[pl.ds(_NS * _RS, _TAIL)],
                            den_out.at[c, pl.ds(_NS * _RS, _TAIL)])

    return sck


_sc128 = _build_sc(128, 8)
_sc48 = _build_sc(_C_PAD, 1)


def _dup_head_mat(a):
    eye8 = jnp.eye(8, dtype=jnp.float32)
    m8 = (a[:, :, None] * eye8[:, None, :]).reshape(128, 8)
    return jnp.concatenate([m8, m8], axis=1)


def _single_head_mat(a):
    ap = jnp.concatenate([a.reshape(_C, 1),
                          jnp.zeros((_C_PAD - _C, 1), jnp.float32)], axis=0)
    return jnp.tile(ap, (1, 16))


def _row8(v, F):
    return jnp.tile(v.reshape(1, F), (8, 1))


def _impl(x, edge_index, W1, a_src1, a_dst1, b1, g1, be1,
          W2, a_src2, a_dst2, b2, g2, be2, W3, a_src3, a_dst3, b3):
    src = edge_index[0].reshape(_NW, _NCH, _CHUNK)
    dst = edge_index[1].reshape(_NW, _NCH, _CHUNK)
    bn_s = 1.0 / jnp.sqrt(jnp.float32(1.0 + 1e-5))

    As1 = _dup_head_mat(a_src1)
    Ad1 = _dup_head_mat(a_dst1)
    As2 = _dup_head_mat(a_src2)
    Ad2 = _dup_head_mat(a_dst2)
    As3 = _single_head_mat(a_src3)
    Ad3 = _single_head_mat(a_dst3)
    W3p = jnp.concatenate([W3, jnp.zeros((128, _C_PAD - _C), jnp.float32)], 1)
    b3p = jnp.concatenate([b3, jnp.zeros((_C_PAD - _C,), jnp.float32)])

    idx16 = jnp.arange(16, dtype=jnp.int32)
    E8 = (idx16[:, None] == (jnp.arange(128, dtype=jnp.int32) // 16)[None, :])
    E8 = E8.astype(jnp.float32)
    E3 = (idx16[:, None] == 0).astype(jnp.float32) * jnp.ones((1, _C_PAD), jnp.float32)

    zrow = jnp.zeros((_RS, 128), jnp.float32)
    zrow3 = jnp.zeros((_RS, _C_PAD), jnp.float32)
    zden = jnp.zeros((_RS, 16), jnp.float32)

    h1, asp1, adp1 = _pre1(x, W1, As1, Ad1)
    acc1, den1 = _sc128(src, dst, h1, asp1, adp1, zrow, zden)
    h2, asp2, adp2 = _combine_pre(
        acc1, den1, h1, asp1, adp1,
        _row8(b1, 128), _row8(g1 * bn_s, 128), _row8(be1, 128),
        E8, W2, As2, Ad2, 128)
    acc2, den2 = _sc128(src, dst, h2, asp2, adp2, zrow, zden)
    h3, asp3, adp3 = _combine_pre(
        acc2, den2, h2, asp2, adp2,
        _row8(b2, 128), _row8(g2 * bn_s, 128), _row8(be2, 128),
        E8, W3p, As3, Ad3, _C_PAD)
    acc3, den3 = _sc48(src, dst, h3, asp3, adp3, zrow3, zden)
    outp = _final(acc3, den3, h3, asp3, adp3, _row8(b3p, _C_PAD), E3)
    return outp[:, :_C]


_impl_jit = jax.jit(_impl)


def kernel(x, edge_index, W1, a_src1, a_dst1, b1, g1, be1,
           W2, a_src2, a_dst2, b2, g2, be2, W3, a_src3, a_dst3, b3):
    return _impl_jit(x, edge_index, W1, a_src1, a_dst1, b1, g1, be1,
                     W2, a_src2, a_dst2, b2, g2, be2, W3, a_src3, a_dst3, b3)
